# baseline (speedup 1.0000x reference)
import jax
import jax.numpy as jnp
from jax import lax
from jax.experimental import pallas as pl
from jax.experimental.pallas import tpu as pltpu
from jax.experimental.pallas import tpu_sc as plsc

N = 10000
E = 320000
D = 128
OUT = 128

NC = 2
NS = 16
NW = NC * NS
EPW = 10240
EPAD = NW * EPW - E
CH = 64
NCHUNK = EPW // CH
NBUF = 4
WSCALE = 16777216.0
RPT_A = 632
RPT_LAST = N - (NS - 1) * RPT_A


def _lane_splat(vec16, j):
    idx = jnp.full((16, 1), j, jnp.int32)
    return lax.gather(
        vec16, idx,
        lax.GatherDimensionNumbers(offset_dims=(), collapsed_slice_dims=(0,),
                                   start_index_map=(0,)),
        (1,), mode=lax.GatherScatterMode.PROMISE_IN_BOUNDS)


def _spmm_body(table, idx3, zed, out,
               i0, i1, i2, i3, r0_, r1, r2, r3,
               acc_sh, si0, si1, si2, si3,
               sg0, sg1, sg2, sg3, ss0, ss1, ss2, ss3):
    idxb = (i0, i1, i2, i3)
    rows = (r0_, r1, r2, r3)
    sem_i = (si0, si1, si2, si3)
    sem_g = (sg0, sg1, sg2, sg3)
    sem_s = (ss0, ss1, ss2, ss3)
    cid = lax.axis_index("c")
    sid = lax.axis_index("s")
    wid = sid * NC + cid

    zr0 = pl.multiple_of(sid * RPT_A, 8)

    @pl.when(sid < NS - 1)
    def _zero_main():
        pltpu.sync_copy(zed.at[pl.ds(zr0, RPT_A)], acc_sh.at[pl.ds(zr0, RPT_A)])

    @pl.when(sid == NS - 1)
    def _zero_last():
        pltpu.sync_copy(zed.at[pl.ds((NS - 1) * RPT_A, RPT_LAST)],
                        acc_sh.at[pl.ds((NS - 1) * RPT_A, RPT_LAST)])

    plsc.subcore_barrier()

    def _issue_idx(m, s):
        pltpu.async_copy(idx3.at[wid, m], idxb[s], sem_i[s])

    def _wait_idx(m, s):
        pltpu.make_async_copy(idx3.at[wid, m], idxb[s], sem_i[s]).wait()

    def _issue_gather(s):
        pltpu.async_copy(table.at[idxb[s].at[0]], rows[s], sem_g[s])

    def _wait_gather(s):
        pltpu.make_async_copy(table.at[idxb[s].at[0]], rows[s],
                              sem_g[s]).wait()

    def _scatter(s):
        pltpu.async_copy(rows[s], acc_sh.at[idxb[s].at[1]], sem_s[s],
                         add=True)

    def _drain_scatter(s):
        pltpu.make_async_copy(rows[s], acc_sh.at[idxb[s].at[1]],
                              sem_s[s]).wait()

    def _multiply(s):
        for g in range(CH // 16):
            w16 = idxb[s][2, pl.ds(g * 16, 16)].astype(jnp.float32) * (
                1.0 / WSCALE)
            for j in range(16):
                wspl = _lane_splat(w16, j)
                e = g * 16 + j
                for dd in range(D // 16):
                    sl = pl.ds(dd * 16, 16)
                    rows[s][e, sl] = rows[s][e, sl] * wspl

    for k in range(NBUF):
        _issue_idx(k, k)
    for k in range(NBUF - 2):
        _wait_idx(k, k)
        _issue_gather(k)

    def ring_body(p, carry):
        for b in range(NBUF):
            m = p * NBUF + b
            _wait_gather(b)
            _multiply(b)
            _scatter(b)

            @pl.when(m + NBUF < NCHUNK)
            def _issue_next_idx():
                _issue_idx(m + NBUF, b)

            @pl.when(m + NBUF - 2 < NCHUNK)
            def _issue_next_gather():
                s3 = (b + NBUF - 2) % NBUF

                @pl.when(m >= 2)
                def _drain_prev_scatter():
                    _drain_scatter(s3)

                _wait_idx(m + NBUF - 2, s3)
                _issue_gather(s3)

        return carry

    lax.fori_loop(0, NCHUNK // NBUF, ring_body, 0)

    for b in range(NBUF):
        _drain_scatter(b)

    plsc.subcore_barrier()

    @pl.when(sid < NS - 1)
    def _dump_main():
        pltpu.sync_copy(acc_sh.at[pl.ds(zr0, RPT_A)],
                        out.at[cid, pl.ds(zr0, RPT_A)])

    @pl.when(sid == NS - 1)
    def _dump_last():
        pltpu.sync_copy(acc_sh.at[pl.ds((NS - 1) * RPT_A, RPT_LAST)],
                        out.at[cid, pl.ds((NS - 1) * RPT_A, RPT_LAST)])


@jax.jit
def _spmm_partials(table, idx3, zed):
    mesh = plsc.VectorSubcoreMesh(core_axis_name="c", subcore_axis_name="s")
    return pl.kernel(
        _spmm_body,
        mesh=mesh,
        out_type=jax.ShapeDtypeStruct((NC, N, D), jnp.float32),
        scratch_types=(
            [pltpu.VMEM((3, CH), jnp.int32) for _ in range(NBUF)]
            + [pltpu.VMEM((CH, D), jnp.float32) for _ in range(NBUF)]
            + [pltpu.VMEM_SHARED((N, D), jnp.float32)]
            + [pltpu.SemaphoreType.DMA for _ in range(3 * NBUF)]
        ),
    )(table, idx3, zed)


def _combine_body(a_ref, b_ref, o_ref):
    o_ref[...] = a_ref[...] + b_ref[...]


@jax.jit
def _combine(a, b):
    blk = 2000
    grid = N // blk
    return pl.pallas_call(
        _combine_body,
        grid=(grid,),
        in_specs=[pl.BlockSpec((blk, D), lambda i: (i, 0)),
                  pl.BlockSpec((blk, D), lambda i: (i, 0))],
        out_specs=pl.BlockSpec((blk, D), lambda i: (i, 0)),
        out_shape=jax.ShapeDtypeStruct((N, D), jnp.float32),
    )(a, b)


def _final_body(x_ref, y1_ref, y2_ref, a_ref, b_ref, w_ref, bias_ref, o_ref):
    w = w_ref[...]
    acc = jnp.dot(x_ref[...], w[0:D], preferred_element_type=jnp.float32)
    acc += jnp.dot(y1_ref[...], w[D:2 * D], preferred_element_type=jnp.float32)
    acc += jnp.dot(y2_ref[...], w[2 * D:3 * D], preferred_element_type=jnp.float32)
    acc += jnp.dot(a_ref[...] + b_ref[...], w[3 * D:4 * D],
                   preferred_element_type=jnp.float32)
    o_ref[...] = acc + bias_ref[...]


@jax.jit
def _final(x, y1, y2, p3a, p3b, shared_weight, bias2d):
    blk = 2000
    grid = N // blk
    row_spec = pl.BlockSpec((blk, D), lambda i: (i, 0))
    return pl.pallas_call(
        _final_body,
        grid=(grid,),
        in_specs=[row_spec, row_spec, row_spec, row_spec, row_spec,
                  pl.BlockSpec((4 * D, OUT), lambda i: (0, 0)),
                  pl.BlockSpec((1, OUT), lambda i: (0, 0))],
        out_specs=pl.BlockSpec((blk, OUT), lambda i: (i, 0)),
        out_shape=jax.ShapeDtypeStruct((N, OUT), jnp.float32),
    )(x, y1, y2, p3a, p3b, shared_weight, bias2d)


def kernel(x, edge_index, edge_weight, shared_weight, output_bias):
    pad = jnp.zeros((EPAD,), jnp.int32)
    src = jnp.concatenate([edge_index[0], pad]).reshape(NW, NCHUNK, CH)
    dst = jnp.concatenate([edge_index[1], pad]).reshape(NW, NCHUNK, CH)
    wfix = jnp.concatenate(
        [jnp.round(edge_weight * WSCALE).astype(jnp.int32), pad]).reshape(
            NW, NCHUNK, CH)
    idx3 = jnp.stack([src, dst, wfix], axis=2)
    zed = jnp.zeros((N, D), jnp.float32)
    p1 = _spmm_partials(x, idx3, zed)
    y1 = _combine(p1[0], p1[1])
    p2 = _spmm_partials(y1, idx3, zed)
    y2 = _combine(p2[0], p2[1])
    p3 = _spmm_partials(y2, idx3, zed)
    return _final(x, y1, y2, p3[0], p3[1], shared_weight,
                  output_bias.reshape(1, OUT))


# speedup vs baseline: 2.1875x; 2.1875x over previous
import jax
import jax.numpy as jnp
from jax import lax
from jax.experimental import pallas as pl
from jax.experimental.pallas import tpu as pltpu
from jax.experimental.pallas import tpu_sc as plsc

N = 10000
E = 320000
D = 128
OUT = 128

NC = 2
NS = 16
NW = NC * NS
EPW = 10240
EPAD = NW * EPW - E
CH = 64
NCHUNK = EPW // CH
NBUF = 4
WSCALE = 16777216.0
RPT_A = 632
RPT_LAST = N - (NS - 1) * RPT_A


def _lane_splat(vec16, j):
    idx = jnp.full((16, 1), j, jnp.int32)
    return lax.gather(
        vec16, idx,
        lax.GatherDimensionNumbers(offset_dims=(), collapsed_slice_dims=(0,),
                                   start_index_map=(0,)),
        (1,), mode=lax.GatherScatterMode.PROMISE_IN_BOUNDS)


def _spmm_body(table, idx3, zed, out,
               i0, i1, i2, i3, r0_, r1, r2, r3,
               acc_sh, si0, si1, si2, si3,
               sg0, sg1, sg2, sg3, ss0, ss1, ss2, ss3):
    idxb = (i0, i1, i2, i3)
    rows = (r0_, r1, r2, r3)
    sem_i = (si0, si1, si2, si3)
    sem_g = (sg0, sg1, sg2, sg3)
    sem_s = (ss0, ss1, ss2, ss3)
    cid = lax.axis_index("c")
    sid = lax.axis_index("s")
    wid = sid * NC + cid

    zr0 = pl.multiple_of(sid * RPT_A, 8)

    @pl.when(sid < NS - 1)
    def _zero_main():
        pltpu.sync_copy(zed.at[pl.ds(zr0, RPT_A)], acc_sh.at[pl.ds(zr0, RPT_A)])

    @pl.when(sid == NS - 1)
    def _zero_last():
        pltpu.sync_copy(zed.at[pl.ds((NS - 1) * RPT_A, RPT_LAST)],
                        acc_sh.at[pl.ds((NS - 1) * RPT_A, RPT_LAST)])

    plsc.subcore_barrier()

    def _issue_idx(m, s):
        pltpu.async_copy(idx3.at[wid, m], idxb[s], sem_i[s])

    def _wait_idx(m, s):
        pltpu.make_async_copy(idx3.at[wid, m], idxb[s], sem_i[s]).wait()

    def _issue_gather(s):
        pltpu.async_copy(table.at[idxb[s].at[0]], rows[s], sem_g[s])

    def _wait_gather(s):
        pltpu.make_async_copy(table.at[idxb[s].at[0]], rows[s],
                              sem_g[s]).wait()

    def _scatter(s):
        pltpu.async_copy(rows[s], acc_sh.at[idxb[s].at[1]], sem_s[s],
                         add=True)

    def _drain_scatter(s):
        pltpu.make_async_copy(rows[s], acc_sh.at[idxb[s].at[1]],
                              sem_s[s]).wait()

    def _multiply(s):
        for g in range(CH // 16):
            w16 = idxb[s][2, pl.ds(g * 16, 16)].astype(jnp.float32) * (
                1.0 / WSCALE)
            for j in range(16):
                wspl = _lane_splat(w16, j)
                e = g * 16 + j
                for dd in range(D // 16):
                    sl = pl.ds(dd * 16, 16)
                    rows[s][e, sl] = rows[s][e, sl] * wspl

    for k in range(NBUF):
        _issue_idx(k, k)
    for k in range(NBUF - 2):
        _wait_idx(k, k)
        _issue_gather(k)

    def ring_body(p, carry):
        for b in range(NBUF):
            m = p * NBUF + b
            _wait_gather(b)
            _multiply(b)
            _scatter(b)

            @pl.when(m + NBUF < NCHUNK)
            def _issue_next_idx():
                _issue_idx(m + NBUF, b)

            @pl.when(m + NBUF - 2 < NCHUNK)
            def _issue_next_gather():
                s3 = (b + NBUF - 2) % NBUF

                @pl.when(m >= 2)
                def _drain_prev_scatter():
                    _drain_scatter(s3)

                _wait_idx(m + NBUF - 2, s3)
                _issue_gather(s3)

        return carry

    lax.fori_loop(0, NCHUNK // NBUF, ring_body, 0)

    for b in range(NBUF):
        _drain_scatter(b)

    plsc.subcore_barrier()

    @pl.when(sid < NS - 1)
    def _dump_main():
        pltpu.sync_copy(acc_sh.at[pl.ds(zr0, RPT_A)],
                        out.at[cid, pl.ds(zr0, RPT_A)])

    @pl.when(sid == NS - 1)
    def _dump_last():
        pltpu.sync_copy(acc_sh.at[pl.ds((NS - 1) * RPT_A, RPT_LAST)],
                        out.at[cid, pl.ds((NS - 1) * RPT_A, RPT_LAST)])


@jax.jit
def _spmm_partials(table, idx3, zed):
    mesh = plsc.VectorSubcoreMesh(core_axis_name="c", subcore_axis_name="s")
    return pl.kernel(
        _spmm_body,
        mesh=mesh,
        out_type=jax.ShapeDtypeStruct((NC, N, D), jnp.float32),
        scratch_types=(
            [pltpu.VMEM((3, CH), jnp.int32) for _ in range(NBUF)]
            + [pltpu.VMEM((CH, D), jnp.float32) for _ in range(NBUF)]
            + [pltpu.VMEM_SHARED((N, D), jnp.float32)]
            + [pltpu.SemaphoreType.DMA for _ in range(3 * NBUF)]
        ),
    )(table, idx3, zed)


def _combine_body(a_ref, b_ref, o_ref):
    o_ref[...] = a_ref[...] + b_ref[...]


@jax.jit
def _combine(a, b):
    blk = 2000
    grid = N // blk
    return pl.pallas_call(
        _combine_body,
        grid=(grid,),
        in_specs=[pl.BlockSpec((blk, D), lambda i: (i, 0)),
                  pl.BlockSpec((blk, D), lambda i: (i, 0))],
        out_specs=pl.BlockSpec((blk, D), lambda i: (i, 0)),
        out_shape=jax.ShapeDtypeStruct((N, D), jnp.float32),
    )(a, b)


def _final_body(x_ref, y1_ref, y2_ref, a_ref, b_ref, w_ref, bias_ref, o_ref):
    w = w_ref[...]
    acc = jnp.dot(x_ref[...], w[0:D], preferred_element_type=jnp.float32)
    acc += jnp.dot(y1_ref[...], w[D:2 * D], preferred_element_type=jnp.float32)
    acc += jnp.dot(y2_ref[...], w[2 * D:3 * D], preferred_element_type=jnp.float32)
    acc += jnp.dot(a_ref[...] + b_ref[...], w[3 * D:4 * D],
                   preferred_element_type=jnp.float32)
    o_ref[...] = acc + bias_ref[...]


@jax.jit
def _final(x, y1, y2, p3a, p3b, shared_weight, bias2d):
    blk = 2000
    grid = N // blk
    row_spec = pl.BlockSpec((blk, D), lambda i: (i, 0))
    return pl.pallas_call(
        _final_body,
        grid=(grid,),
        in_specs=[row_spec, row_spec, row_spec, row_spec, row_spec,
                  pl.BlockSpec((4 * D, OUT), lambda i: (0, 0)),
                  pl.BlockSpec((1, OUT), lambda i: (0, 0))],
        out_specs=pl.BlockSpec((blk, OUT), lambda i: (i, 0)),
        out_shape=jax.ShapeDtypeStruct((N, OUT), jnp.float32),
    )(x, y1, y2, p3a, p3b, shared_weight, bias2d)


def kernel(x, edge_index, edge_weight, shared_weight, output_bias):
    pad_ix = (jnp.arange(EPAD, dtype=jnp.int32) * 16) % N
    pad_w = jnp.zeros((EPAD,), jnp.int32)
    src = jnp.concatenate([edge_index[0], pad_ix]).reshape(NW, NCHUNK, CH)
    dst = jnp.concatenate([edge_index[1], pad_ix]).reshape(NW, NCHUNK, CH)
    wfix = jnp.concatenate(
        [jnp.round(edge_weight * WSCALE).astype(jnp.int32), pad_w]).reshape(
            NW, NCHUNK, CH)
    idx3 = jnp.stack([src, dst, wfix], axis=2)
    zed = jnp.zeros((N, D), jnp.float32)
    p1 = _spmm_partials(x, idx3, zed)
    y1 = _combine(p1[0], p1[1])
    p2 = _spmm_partials(y1, idx3, zed)
    y2 = _combine(p2[0], p2[1])
    p3 = _spmm_partials(y2, idx3, zed)
    return _final(x, y1, y2, p3[0], p3[1], shared_weight,
                  output_bias.reshape(1, OUT))


# speedup vs baseline: 2.2457x; 1.0266x over previous
import jax
import jax.numpy as jnp
from jax import lax
from jax.experimental import pallas as pl
from jax.experimental.pallas import tpu as pltpu
from jax.experimental.pallas import tpu_sc as plsc

N = 10000
E = 320000
D = 128
OUT = 128

NC = 2
NS = 16
NW = NC * NS
EPW = E // NW
CH = 40
NCHUNK = EPW // CH
NBUF = 5
WSCALE = 16777216.0
RPT_A = 632
RPT_LAST = N - (NS - 1) * RPT_A


def _lane_splat(vec16, j):
    idx = jnp.full((16, 1), j, jnp.int32)
    return lax.gather(
        vec16, idx,
        lax.GatherDimensionNumbers(offset_dims=(), collapsed_slice_dims=(0,),
                                   start_index_map=(0,)),
        (1,), mode=lax.GatherScatterMode.PROMISE_IN_BOUNDS)


def _spmm_body(table, idx3, zed, out,
               i0, i1, i2, i3, i4, r0_, r1, r2, r3, r4,
               acc_sh, si0, si1, si2, si3, si4,
               sg0, sg1, sg2, sg3, sg4, ss0, ss1, ss2, ss3, ss4):
    idxb = (i0, i1, i2, i3, i4)
    rows = (r0_, r1, r2, r3, r4)
    sem_i = (si0, si1, si2, si3, si4)
    sem_g = (sg0, sg1, sg2, sg3, sg4)
    sem_s = (ss0, ss1, ss2, ss3, ss4)
    cid = lax.axis_index("c")
    sid = lax.axis_index("s")
    wid = sid * NC + cid

    zr0 = pl.multiple_of(sid * RPT_A, 8)

    @pl.when(sid < NS - 1)
    def _zero_main():
        pltpu.sync_copy(zed.at[pl.ds(zr0, RPT_A)], acc_sh.at[pl.ds(zr0, RPT_A)])

    @pl.when(sid == NS - 1)
    def _zero_last():
        pltpu.sync_copy(zed.at[pl.ds((NS - 1) * RPT_A, RPT_LAST)],
                        acc_sh.at[pl.ds((NS - 1) * RPT_A, RPT_LAST)])

    plsc.subcore_barrier()

    def _issue_idx(m, s):
        pltpu.async_copy(idx3.at[wid, m], idxb[s], sem_i[s])

    def _wait_idx(m, s):
        pltpu.make_async_copy(idx3.at[wid, m], idxb[s], sem_i[s]).wait()

    def _issue_gather(s):
        pltpu.async_copy(table.at[idxb[s].at[0]], rows[s], sem_g[s])

    def _wait_gather(s):
        pltpu.make_async_copy(table.at[idxb[s].at[0]], rows[s],
                              sem_g[s]).wait()

    def _scatter(s):
        pltpu.async_copy(rows[s], acc_sh.at[idxb[s].at[1]], sem_s[s],
                         add=True)

    def _drain_scatter(s):
        pltpu.make_async_copy(rows[s], acc_sh.at[idxb[s].at[1]],
                              sem_s[s]).wait()

    def _multiply(s):
        for off, jlist in ((0, range(16)), (16, range(16)),
                           (CH - 16, range(32 - (CH - 16), 16))):
            w16 = idxb[s][2, pl.ds(off, 16)].astype(jnp.float32) * (
                1.0 / WSCALE)
            for j in jlist:
                wspl = _lane_splat(w16, j)
                e = off + j
                for dd in range(D // 16):
                    sl = pl.ds(dd * 16, 16)
                    rows[s][e, sl] = rows[s][e, sl] * wspl

    for k in range(NBUF):
        _issue_idx(k, k)
    for k in range(NBUF - 2):
        _wait_idx(k, k)
        _issue_gather(k)

    def ring_body(p, carry):
        for b in range(NBUF):
            m = p * NBUF + b
            _wait_gather(b)
            _multiply(b)
            _scatter(b)

            @pl.when(m + NBUF < NCHUNK)
            def _issue_next_idx():
                _issue_idx(m + NBUF, b)

            @pl.when(m + NBUF - 2 < NCHUNK)
            def _issue_next_gather():
                s3 = (b + NBUF - 2) % NBUF

                @pl.when(m >= 2)
                def _drain_prev_scatter():
                    _drain_scatter(s3)

                _wait_idx(m + NBUF - 2, s3)
                _issue_gather(s3)

        return carry

    lax.fori_loop(0, NCHUNK // NBUF, ring_body, 0)

    for b in range(NBUF):
        _drain_scatter(b)

    plsc.subcore_barrier()

    @pl.when(sid < NS - 1)
    def _dump_main():
        pltpu.sync_copy(acc_sh.at[pl.ds(zr0, RPT_A)],
                        out.at[cid, pl.ds(zr0, RPT_A)])

    @pl.when(sid == NS - 1)
    def _dump_last():
        pltpu.sync_copy(acc_sh.at[pl.ds((NS - 1) * RPT_A, RPT_LAST)],
                        out.at[cid, pl.ds((NS - 1) * RPT_A, RPT_LAST)])


@jax.jit
def _spmm_partials(table, idx3, zed):
    mesh = plsc.VectorSubcoreMesh(core_axis_name="c", subcore_axis_name="s")
    return pl.kernel(
        _spmm_body,
        mesh=mesh,
        out_type=jax.ShapeDtypeStruct((NC, N, D), jnp.float32),
        scratch_types=(
            [pltpu.VMEM((3, CH), jnp.int32) for _ in range(NBUF)]
            + [pltpu.VMEM((CH, D), jnp.float32) for _ in range(NBUF)]
            + [pltpu.VMEM_SHARED((N, D), jnp.float32)]
            + [pltpu.SemaphoreType.DMA for _ in range(3 * NBUF)]
        ),
    )(table, idx3, zed)


def _combine_body(a_ref, b_ref, o_ref):
    o_ref[...] = a_ref[...] + b_ref[...]


@jax.jit
def _combine(a, b):
    blk = 2000
    grid = N // blk
    return pl.pallas_call(
        _combine_body,
        grid=(grid,),
        in_specs=[pl.BlockSpec((blk, D), lambda i: (i, 0)),
                  pl.BlockSpec((blk, D), lambda i: (i, 0))],
        out_specs=pl.BlockSpec((blk, D), lambda i: (i, 0)),
        out_shape=jax.ShapeDtypeStruct((N, D), jnp.float32),
    )(a, b)


def _final_body(x_ref, y1_ref, y2_ref, a_ref, b_ref, w_ref, bias_ref, o_ref):
    w = w_ref[...]
    acc = jnp.dot(x_ref[...], w[0:D], preferred_element_type=jnp.float32)
    acc += jnp.dot(y1_ref[...], w[D:2 * D], preferred_element_type=jnp.float32)
    acc += jnp.dot(y2_ref[...], w[2 * D:3 * D], preferred_element_type=jnp.float32)
    acc += jnp.dot(a_ref[...] + b_ref[...], w[3 * D:4 * D],
                   preferred_element_type=jnp.float32)
    o_ref[...] = acc + bias_ref[...]


@jax.jit
def _final(x, y1, y2, p3a, p3b, shared_weight, bias2d):
    blk = 2000
    grid = N // blk
    row_spec = pl.BlockSpec((blk, D), lambda i: (i, 0))
    return pl.pallas_call(
        _final_body,
        grid=(grid,),
        in_specs=[row_spec, row_spec, row_spec, row_spec, row_spec,
                  pl.BlockSpec((4 * D, OUT), lambda i: (0, 0)),
                  pl.BlockSpec((1, OUT), lambda i: (0, 0))],
        out_specs=pl.BlockSpec((blk, OUT), lambda i: (i, 0)),
        out_shape=jax.ShapeDtypeStruct((N, OUT), jnp.float32),
    )(x, y1, y2, p3a, p3b, shared_weight, bias2d)


def kernel(x, edge_index, edge_weight, shared_weight, output_bias):
    src = edge_index[0].reshape(NW, NCHUNK, CH)
    dst = edge_index[1].reshape(NW, NCHUNK, CH)
    wfix = jnp.round(edge_weight * WSCALE).astype(jnp.int32).reshape(
        NW, NCHUNK, CH)
    idx3 = jnp.stack([src, dst, wfix], axis=2)
    zed = jnp.zeros((N, D), jnp.float32)
    p1 = _spmm_partials(x, idx3, zed)
    y1 = _combine(p1[0], p1[1])
    p2 = _spmm_partials(y1, idx3, zed)
    y2 = _combine(p2[0], p2[1])
    p3 = _spmm_partials(y2, idx3, zed)
    return _final(x, y1, y2, p3[0], p3[1], shared_weight,
                  output_bias.reshape(1, OUT))


# speedup vs baseline: 2.3591x; 1.0505x over previous
import jax
import jax.numpy as jnp
from jax import lax
from jax.experimental import pallas as pl
from jax.experimental.pallas import tpu as pltpu
from jax.experimental.pallas import tpu_sc as plsc

N = 10000
E = 320000
D = 128
OUT = 128

NC = 2
NS = 16
NW = NC * NS
EPW = E // NW
CH = 40
NCHUNK = EPW // CH
NBUF = 5
RPT_A = 632
RPT_LAST = N - (NS - 1) * RPT_A


def _lane_splat(vec16, j):
    idx = jnp.full((16, 1), j, jnp.int32)
    return lax.gather(
        vec16, idx,
        lax.GatherDimensionNumbers(offset_dims=(), collapsed_slice_dims=(0,),
                                   start_index_map=(0,)),
        (1,), mode=lax.GatherScatterMode.PROMISE_IN_BOUNDS)


def _spmm_body(table, idx3, wsr, zed, out,
               i0, i1, i2, i3, i4, w0, w1, w2, w3, w4, r0_, r1, r2, r3, r4,
               acc_sh, si0, si1, si2, si3, si4, sw0, sw1, sw2, sw3, sw4,
               sg0, sg1, sg2, sg3, sg4, ss0, ss1, ss2, ss3, ss4):
    idxb = (i0, i1, i2, i3, i4)
    wbuf = (w0, w1, w2, w3, w4)
    rows = (r0_, r1, r2, r3, r4)
    sem_i = (si0, si1, si2, si3, si4)
    sem_w = (sw0, sw1, sw2, sw3, sw4)
    sem_g = (sg0, sg1, sg2, sg3, sg4)
    sem_s = (ss0, ss1, ss2, ss3, ss4)
    cid = lax.axis_index("c")
    sid = lax.axis_index("s")
    wid = sid * NC + cid

    zr0 = pl.multiple_of(sid * RPT_A, 8)

    @pl.when(sid < NS - 1)
    def _zero_main():
        pltpu.sync_copy(zed.at[pl.ds(zr0, RPT_A)], acc_sh.at[pl.ds(zr0, RPT_A)])

    @pl.when(sid == NS - 1)
    def _zero_last():
        pltpu.sync_copy(zed.at[pl.ds((NS - 1) * RPT_A, RPT_LAST)],
                        acc_sh.at[pl.ds((NS - 1) * RPT_A, RPT_LAST)])

    plsc.subcore_barrier()

    def _issue_idx(m, s):
        pltpu.async_copy(idx3.at[wid, m], idxb[s], sem_i[s])
        pltpu.async_copy(wsr.at[wid, m], wbuf[s], sem_w[s])

    def _wait_idx(m, s):
        pltpu.make_async_copy(idx3.at[wid, m], idxb[s], sem_i[s]).wait()
        pltpu.make_async_copy(wsr.at[wid, m], wbuf[s], sem_w[s]).wait()

    def _issue_gather(s):
        pltpu.async_copy(table.at[idxb[s].at[0]], rows[s], sem_g[s])

    def _wait_gather(s):
        pltpu.make_async_copy(table.at[idxb[s].at[0]], rows[s],
                              sem_g[s]).wait()

    def _scatter(s):
        pltpu.async_copy(rows[s], acc_sh.at[idxb[s].at[1]], sem_s[s],
                         add=True)

    def _drain_scatter(s):
        pltpu.make_async_copy(rows[s], acc_sh.at[idxb[s].at[1]],
                              sem_s[s]).wait()

    def _multiply(s):
        for off, jlist in ((0, range(16)), (16, range(16)),
                           (CH - 16, range(32 - (CH - 16), 16))):
            w16 = wbuf[s][pl.ds(off, 16)]
            for j in jlist:
                wspl = _lane_splat(w16, j)
                e = off + j
                for dd in range(D // 16):
                    sl = pl.ds(dd * 16, 16)
                    rows[s][e, sl] = rows[s][e, sl] * wspl

    for k in range(NBUF):
        _issue_idx(k, k)
    for k in range(NBUF - 2):
        _wait_idx(k, k)
        _issue_gather(k)

    def ring_body(p, carry):
        for b in range(NBUF):
            m = p * NBUF + b
            _wait_gather(b)
            _multiply(b)
            _scatter(b)

            @pl.when(m + NBUF < NCHUNK)
            def _issue_next_idx():
                _issue_idx(m + NBUF, b)

            @pl.when(m + NBUF - 2 < NCHUNK)
            def _issue_next_gather():
                s3 = (b + NBUF - 2) % NBUF

                @pl.when(m >= 2)
                def _drain_prev_scatter():
                    _drain_scatter(s3)

                _wait_idx(m + NBUF - 2, s3)
                _issue_gather(s3)

        return carry

    lax.fori_loop(0, NCHUNK // NBUF, ring_body, 0)

    for b in range(NBUF):
        _drain_scatter(b)

    plsc.subcore_barrier()

    @pl.when(sid < NS - 1)
    def _dump_main():
        pltpu.sync_copy(acc_sh.at[pl.ds(zr0, RPT_A)],
                        out.at[cid, pl.ds(zr0, RPT_A)])

    @pl.when(sid == NS - 1)
    def _dump_last():
        pltpu.sync_copy(acc_sh.at[pl.ds((NS - 1) * RPT_A, RPT_LAST)],
                        out.at[cid, pl.ds((NS - 1) * RPT_A, RPT_LAST)])


@jax.jit
def _spmm_partials(table, idx3, wsr, zed):
    mesh = plsc.VectorSubcoreMesh(core_axis_name="c", subcore_axis_name="s")
    return pl.kernel(
        _spmm_body,
        mesh=mesh,
        out_type=jax.ShapeDtypeStruct((NC, N, D), jnp.float32),
        scratch_types=(
            [pltpu.VMEM((2, CH), jnp.int32) for _ in range(NBUF)]
            + [pltpu.VMEM((CH,), jnp.float32) for _ in range(NBUF)]
            + [pltpu.VMEM((CH, D), jnp.float32) for _ in range(NBUF)]
            + [pltpu.VMEM_SHARED((N, D), jnp.float32)]
            + [pltpu.SemaphoreType.DMA for _ in range(4 * NBUF)]
        ),
    )(table, idx3, wsr, zed)


def _combine_body(a_ref, b_ref, o_ref):
    o_ref[...] = a_ref[...] + b_ref[...]


@jax.jit
def _combine(a, b):
    blk = 2000
    grid = N // blk
    return pl.pallas_call(
        _combine_body,
        grid=(grid,),
        in_specs=[pl.BlockSpec((blk, D), lambda i: (i, 0)),
                  pl.BlockSpec((blk, D), lambda i: (i, 0))],
        out_specs=pl.BlockSpec((blk, D), lambda i: (i, 0)),
        out_shape=jax.ShapeDtypeStruct((N, D), jnp.float32),
    )(a, b)


def _mm01_body(x_ref, y1_ref, w_ref, bias_ref, o_ref):
    w = w_ref[...]
    acc = jnp.dot(x_ref[...], w[0:D], preferred_element_type=jnp.float32)
    acc += jnp.dot(y1_ref[...], w[D:2 * D], preferred_element_type=jnp.float32)
    o_ref[...] = acc + bias_ref[...]


@jax.jit
def _mm01(x, y1, w01, bias2d):
    blk = 2000
    grid = N // blk
    row_spec = pl.BlockSpec((blk, D), lambda i: (i, 0))
    return pl.pallas_call(
        _mm01_body,
        grid=(grid,),
        in_specs=[row_spec, row_spec,
                  pl.BlockSpec((2 * D, OUT), lambda i: (0, 0)),
                  pl.BlockSpec((1, OUT), lambda i: (0, 0))],
        out_specs=pl.BlockSpec((blk, OUT), lambda i: (i, 0)),
        out_shape=jax.ShapeDtypeStruct((N, OUT), jnp.float32),
    )(x, y1, w01, bias2d)


def _mm2_body(z_ref, y2_ref, w_ref, o_ref):
    o_ref[...] = z_ref[...] + jnp.dot(y2_ref[...], w_ref[...],
                                      preferred_element_type=jnp.float32)


@jax.jit
def _mm2(z, y2, w2):
    blk = 2000
    grid = N // blk
    row_spec = pl.BlockSpec((blk, D), lambda i: (i, 0))
    return pl.pallas_call(
        _mm2_body,
        grid=(grid,),
        in_specs=[row_spec, row_spec,
                  pl.BlockSpec((D, OUT), lambda i: (0, 0))],
        out_specs=pl.BlockSpec((blk, OUT), lambda i: (i, 0)),
        out_shape=jax.ShapeDtypeStruct((N, OUT), jnp.float32),
    )(z, y2, w2)


def _final_body(z_ref, a_ref, b_ref, w_ref, o_ref):
    o_ref[...] = z_ref[...] + jnp.dot(a_ref[...] + b_ref[...], w_ref[...],
                                      preferred_element_type=jnp.float32)


@jax.jit
def _final(z, p3a, p3b, w3):
    blk = 2000
    grid = N // blk
    row_spec = pl.BlockSpec((blk, D), lambda i: (i, 0))
    return pl.pallas_call(
        _final_body,
        grid=(grid,),
        in_specs=[row_spec, row_spec, row_spec,
                  pl.BlockSpec((D, OUT), lambda i: (0, 0))],
        out_specs=pl.BlockSpec((blk, OUT), lambda i: (i, 0)),
        out_shape=jax.ShapeDtypeStruct((N, OUT), jnp.float32),
    )(z, p3a, p3b, w3)


def kernel(x, edge_index, edge_weight, shared_weight, output_bias):
    src = edge_index[0].reshape(NW, NCHUNK, CH)
    dst = edge_index[1].reshape(NW, NCHUNK, CH)
    idx3 = jnp.stack([src, dst], axis=2)
    wsr = edge_weight.reshape(NW, NCHUNK, CH)
    zed = jnp.zeros((N, D), jnp.float32)
    p1 = _spmm_partials(x, idx3, wsr, zed)
    y1 = _combine(p1[0], p1[1])
    p2 = _spmm_partials(y1, idx3, wsr, zed)
    z01 = _mm01(x, y1, shared_weight[0:2 * D], output_bias.reshape(1, OUT))
    y2 = _combine(p2[0], p2[1])
    p3 = _spmm_partials(y2, idx3, wsr, zed)
    z2 = _mm2(z01, y2, shared_weight[2 * D:3 * D])
    return _final(z2, p3[0], p3[1], shared_weight[3 * D:4 * D])
